# Initial kernel scaffold; baseline (speedup 1.0000x reference)
import jax
import jax.numpy as jnp
from jax.experimental import pallas as pl


def kernel(output, mask, ind, target):
    raise NotImplementedError("write your pallas kernel here")


# speedup vs baseline: 2.1316x; 2.1316x over previous
import functools

import jax
import jax.numpy as jnp
from jax import lax
from jax.experimental import pallas as pl
from jax.experimental.pallas import tpu as pltpu
from jax.experimental.pallas import tpu_sc as plsc

B, C, H, W = 64, 2, 128, 128
HW = H * W
K = 500
KP = 512
L = 16
NC, NS = 2, 16
NW = NC * NS
BPW = B // NW


def _make_sc_loss():
    mesh = plsc.VectorSubcoreMesh(core_axis_name="c", subcore_axis_name="s")

    @functools.partial(
        pl.kernel,
        out_type=jax.ShapeDtypeStruct((NC, 2 * L), jnp.float32),
        mesh=mesh,
        compiler_params=pltpu.CompilerParams(needs_layout_passes=False),
        scratch_types=[
            pltpu.VMEM((C * HW,), jnp.float32),
            pltpu.VMEM((C * HW,), jnp.float32),
            pltpu.VMEM((BPW, KP), jnp.int32),
            pltpu.VMEM((BPW, KP), jnp.float32),
            pltpu.VMEM((BPW, C, KP), jnp.float32),
            pltpu.VMEM((2 * L,), jnp.float32),
            pltpu.VMEM((NS, 2 * L), jnp.float32),
            pltpu.VMEM_SHARED((NS, 2 * L), jnp.float32),
            pltpu.SemaphoreType.DMA,
            pltpu.SemaphoreType.DMA,
        ],
    )
    def sc_loss(feat_hbm, ind_hbm, mask_hbm, tgt_hbm, out_hbm,
                feat0_v, feat1_v, ind_v, mask_v, tgt_v,
                stage_v, red_v, shared, sem0, sem1):
        cid = lax.axis_index("c")
        sid = lax.axis_index("s")
        wid = sid * NC + cid
        b0 = wid * BPW

        cp0 = pltpu.async_copy(feat_hbm.at[b0], feat0_v, sem0)
        cp1 = pltpu.async_copy(feat_hbm.at[b0 + 1], feat1_v, sem1)
        pltpu.sync_copy(ind_hbm.at[pl.ds(b0, BPW)], ind_v)
        pltpu.sync_copy(mask_hbm.at[pl.ds(b0, BPW)], mask_v)
        pltpu.sync_copy(tgt_hbm.at[pl.ds(b0, BPW)], tgt_v)
        cp0.wait()
        cp1.wait()

        num = jnp.zeros((L,), jnp.float32)
        msum = jnp.zeros((L,), jnp.float32)
        for i, feat_v in ((0, feat0_v), (1, feat1_v)):
            for j in range(KP // L):
                s = j * L
                idx = ind_v[i, pl.ds(s, L)]
                m = mask_v[i, pl.ds(s, L)]
                x0 = plsc.load_gather(feat_v, [idx])
                x1 = plsc.load_gather(feat_v, [idx + HW])
                t0 = tgt_v[i, 0, pl.ds(s, L)]
                t1 = tgt_v[i, 1, pl.ds(s, L)]
                num = num + m * (jnp.abs(x0 - t0) + jnp.abs(x1 - t1))
                msum = msum + m

        stage_v[pl.ds(0, L)] = num
        stage_v[pl.ds(L, L)] = msum
        pltpu.sync_copy(stage_v, shared.at[sid])
        plsc.subcore_barrier()

        @pl.when(sid == 0)
        def _():
            pltpu.sync_copy(shared, red_v)
            acc_n = jnp.zeros((L,), jnp.float32)
            acc_m = jnp.zeros((L,), jnp.float32)
            for t in range(NS):
                acc_n = acc_n + red_v[t, pl.ds(0, L)]
                acc_m = acc_m + red_v[t, pl.ds(L, L)]
            n_s = jnp.sum(acc_n)
            m_s = jnp.sum(acc_m)
            stage_v[pl.ds(0, L)] = jnp.broadcast_to(n_s, (L,))
            stage_v[pl.ds(L, L)] = jnp.broadcast_to(m_s, (L,))
            pltpu.sync_copy(stage_v, out_hbm.at[cid])

    return sc_loss


_SC_LOSS = _make_sc_loss()


def kernel(output, mask, ind, target):
    feat = output.reshape(B, C * HW).astype(jnp.float32)
    ind_p = jnp.pad(ind.astype(jnp.int32), ((0, 0), (0, KP - K)))
    mask_p = jnp.pad(mask.astype(jnp.float32), ((0, 0), (0, KP - K)))
    tgt_p = jnp.pad(jnp.transpose(target.astype(jnp.float32), (0, 2, 1)),
                    ((0, 0), (0, 0), (0, KP - K)))
    res = _SC_LOSS(feat, ind_p, mask_p, tgt_p)
    num = res[0, 0] + res[1, 0]
    msum = res[0, L] + res[1, L]
    return num / (C * msum + 0.0001)
